# baseline (speedup 1.0000x reference)
import jax
import jax.numpy as jnp
from jax.experimental import pallas as pl
from jax.experimental.pallas import tpu as pltpu

BS, S, C, IN_DIM, H = 64, 60, 19, 50, 128
L_LOC, L_GLOB, DEPTH, KNN, P, NCLS = 2, 2, 2, 5, 8, 4
DECAY = 0.9
N = C * P
SP = 64
R = C * SP
GB = 16
SAMP = 4
RL = SAMP * R
CP = 24
RSQ_H = float(1.0 / (128.0 ** 0.5))


def _ln(x, g, b):
    m = jnp.mean(x, axis=-1, keepdims=True)
    xc = x - m
    v = jnp.mean(xc * xc, axis=-1, keepdims=True)
    return xc * jax.lax.rsqrt(v + 1e-5) * g + b


def _softmax_rows(logits):
    m = jnp.max(logits, axis=-1, keepdims=True)
    e = jnp.exp(logits - m)
    return e / jnp.sum(e, axis=-1, keepdims=True)


def _dot(a, b):
    return jnp.dot(a, b, preferred_element_type=jnp.float32)


def _dot_t(a, b):
    return jax.lax.dot_general(a, b, (((1,), (1,)), ((), ())),
                               preferred_element_type=jnp.float32)


def _dot_lt(a, b):
    return jax.lax.dot_general(a, b, (((0,), (0,)), ((), ())),
                               preferred_element_type=jnp.float32)


def _local_kernel(x_ref, wfc_ref, bfc_ref, lwq_ref, lpet_ref, lw_ref,
                  lg_ref, lb_ref, wp_ref, prox_ref, out_ref,
                  h_ref, q_ref, a_ref, m_ref, pls_ref):
    row = jax.lax.broadcasted_iota(jnp.int32, (RL, 1), 0)
    rmask = (row % SP) < S
    h_ref[:] = jnp.zeros((RL, H), jnp.float32)
    wfc = wfc_ref[:]
    bfc = bfc_ref[:]
    for s in range(SAMP):
        xw = x_ref[s]
        for c in range(C):
            o = (s * C + c) * SP
            xc = xw[:, c * IN_DIM:(c + 1) * IN_DIM]
            h_ref[o:o + S, :] = _dot(xc, wfc) + bfc

    col = jax.lax.broadcasted_iota(jnp.int32, (RL, SP), 1)
    cmask = col < S
    for l in range(L_LOC):
        q_ref[:] = _dot(h_ref[:] + lpet_ref[l], lwq_ref[l])

        for g in range(SAMP * C):
            qc = q_ref[g * SP:(g + 1) * SP, :]
            a_ref[g * SP:(g + 1) * SP, :] = _dot_t(qc, qc)

        CH = 4 * SP
        cm = cmask[:CH, :]
        for k in range(SAMP * C // 4):
            ck = slice(k * CH, (k + 1) * CH)
            sim = jnp.where(cm, a_ref[ck, :] * RSQ_H, jnp.float32(-1e9))
            work = sim
            for _ in range(KNN - 1):
                mx = jnp.max(work, axis=-1, keepdims=True)
                work = jnp.where(work >= mx, jnp.float32(-1e30), work)
            thr = jnp.max(work, axis=-1, keepdims=True)
            a_ref[ck, :] = _softmax_rows(
                jnp.where(sim >= thr, sim, jnp.float32(-1e9)))

        out = h_ref[:]
        q_ref[:] = out
        dk = 1.0
        for d in range(DEPTH):
            for g in range(SAMP * C):
                sl = slice(g * SP, (g + 1) * SP)
                m_ref[sl, :] = _dot(a_ref[sl, :], q_ref[sl, :])
            cur = jnp.maximum(_dot(m_ref[:], lw_ref[l, d]), 0.0)
            q_ref[:] = cur
            dk *= DECAY
            out = out + dk * cur
        h_ref[:] = jnp.where(rmask, _ln(out, lg_ref[l], lb_ref[l]), 0.0)

    plog = _dot_t(_dot(h_ref[:], wp_ref[:]), prox_ref[:]) * RSQ_H
    pls_ref[:] = jnp.where(rmask, plog, jnp.float32(-1e9))

    for s in range(SAMP):
        for c in range(C):
            g = s * C + c
            sl = slice(g * SP, (g + 1) * SP)
            plc = pls_ref[sl, :]
            pm = jnp.max(plc, axis=0, keepdims=True)
            ex = jnp.exp(plc - pm)
            sc = ex / jnp.sum(ex, axis=0, keepdims=True)
            out_ref[s, c * P:(c + 1) * P, :] = _dot_lt(sc, h_ref[sl, :])


def _global_kernel(hn_ref, gwq_ref, gpet_ref, gw_ref, gg_ref, gb_ref,
                   fw1_ref, fb1_ref, fw2_ref, fb2_ref, fg_ref, fb_ref,
                   out_ref, q_ref, a_ref, m_ref):
    hn = hn_ref[0]
    for l in range(L_GLOB):
        q_ref[:] = _dot(hn + gpet_ref[l], gwq_ref[l])

        for s in range(GB):
            qs = q_ref[s * N:(s + 1) * N, :]
            a_ref[s * N:(s + 1) * N, :] = _dot_t(qs, qs)

        a_ref[:] = _softmax_rows(a_ref[:] * RSQ_H)
        out = hn
        q_ref[:] = hn
        for d in range(DEPTH):
            for s in range(GB):
                sl = slice(s * N, (s + 1) * N)
                m_ref[sl, :] = _dot(a_ref[sl, :], q_ref[sl, :])
            cur = jnp.maximum(_dot(m_ref[:], gw_ref[l, d]), 0.0)
            q_ref[:] = cur
            out = out + cur
        hn = _ln(out, gg_ref[l], gb_ref[l])
    ffn = _dot(jax.nn.gelu(_dot(hn, fw1_ref[:]) + fb1_ref[:]), fw2_ref[:])
    m_ref[:] = _ln(hn + ffn + fb2_ref[:], fg_ref[:], fb_ref[:])

    ii = jax.lax.broadcasted_iota(jnp.int32, (CP, N), 0)
    jj = jax.lax.broadcasted_iota(jnp.int32, (CP, N), 1)
    grp = jnp.where((jj // P == ii) & (ii < C), jnp.float32(1.0 / P),
                    jnp.float32(0.0))

    for s in range(GB):
        zs = m_ref[s * N:(s + 1) * N, :]
        out_ref[0, s * CP:(s + 1) * CP, :] = jnp.maximum(_dot(grp, zs), 0.0)


def _dec_kernel(z_ref, w1_ref, b1_ref, w2_ref, b2_ref, out_ref):
    a = jnp.maximum(_dot(z_ref[:], w1_ref[:]) + b1_ref[:], 0.0)
    out_ref[:, :] = _dot(a, w2_ref[:]) + b2_ref[:]


def _full(shape):
    nd = len(shape)
    return pl.BlockSpec(shape, lambda *_, _nd=nd: (0,) * _nd)


def kernel(x, p, y, params):
    f32 = jnp.float32
    xf = x.reshape(BS, S, C * IN_DIM)
    lpet = jnp.tile(jnp.pad(params['loc_pe'], ((0, 0), (0, SP - S), (0, 0))),
                    (1, SAMP * C, 1))
    ws1 = (params['W_fc'], params['b_fc'].reshape(1, H), params['loc_Wq'],
           lpet, params['loc_W'], params['loc_ln_g'].reshape(L_LOC, 1, H),
           params['loc_ln_b'].reshape(L_LOC, 1, H), params['Wp'],
           params['proxies'])
    in_specs = [pl.BlockSpec((SAMP, S, C * IN_DIM), lambda b: (b, 0, 0))]
    in_specs += [_full(w.shape) for w in ws1]
    pooled = pl.pallas_call(
        _local_kernel,
        grid=(BS // SAMP,),
        in_specs=in_specs,
        out_specs=pl.BlockSpec((SAMP, N, H), lambda b: (b, 0, 0)),
        out_shape=jax.ShapeDtypeStruct((BS, N, H), f32),
        scratch_shapes=[pltpu.VMEM((RL, H), f32), pltpu.VMEM((RL, H), f32),
                        pltpu.VMEM((RL, SP), f32), pltpu.VMEM((RL, H), f32),
                        pltpu.VMEM((RL, P), f32)],
        compiler_params=pltpu.CompilerParams(
            dimension_semantics=("arbitrary",)),
    )(xf, *ws1)

    hn_in = pooled.reshape(BS // GB, GB * N, H)
    gpet = jnp.tile(params['glob_pe'], (1, GB, 1))
    ws2 = (params['glob_Wq'], gpet, params['glob_W'],
           params['glob_ln_g'].reshape(L_GLOB, 1, H),
           params['glob_ln_b'].reshape(L_GLOB, 1, H),
           params['ffn_W1'], params['ffn_b1'].reshape(1, 4 * H),
           params['ffn_W2'], params['ffn_b2'].reshape(1, H),
           params['ffn_ln_g'].reshape(1, H), params['ffn_ln_b'].reshape(1, H))
    in_specs2 = [pl.BlockSpec((1, GB * N, H), lambda b: (b, 0, 0))]
    in_specs2 += [_full(w.shape) for w in ws2]
    z = pl.pallas_call(
        _global_kernel,
        grid=(BS // GB,),
        in_specs=in_specs2,
        out_specs=pl.BlockSpec((1, GB * CP, H), lambda b: (b, 0, 0)),
        out_shape=jax.ShapeDtypeStruct((BS // GB, GB * CP, H), f32),
        scratch_shapes=[pltpu.VMEM((GB * N, H), f32),
                        pltpu.VMEM((GB * N, N), f32),
                        pltpu.VMEM((GB * N, H), f32)],
        compiler_params=pltpu.CompilerParams(
            dimension_semantics=("arbitrary",)),
    )(hn_in, *ws2)

    zf = z.reshape(BS, CP, H)[:, :C, :].reshape(BS, C * H)
    return pl.pallas_call(
        _dec_kernel,
        in_specs=[_full((BS, C * H)), _full((C * H, H)), _full((1, H)),
                  _full((H, NCLS)), _full((1, NCLS))],
        out_specs=_full((BS, NCLS)),
        out_shape=jax.ShapeDtypeStruct((BS, NCLS), f32),
    )(zf, params['dec_W1'], params['dec_b1'].reshape(1, H),
      params['dec_W2'], params['dec_b2'].reshape(1, NCLS))


# speedup vs baseline: 1.3812x; 1.3812x over previous
import jax
import jax.numpy as jnp
from jax.experimental import pallas as pl
from jax.experimental.pallas import tpu as pltpu

BS, S, C, IN_DIM, H = 64, 60, 19, 50, 128
L_LOC, L_GLOB, DEPTH, KNN, P, NCLS = 2, 2, 2, 5, 8, 4
DECAY = 0.9
N = C * P
SP = 64
R = C * SP
GB = 16
SAMP = 4
RL = SAMP * R
CP = 24
RSQ_H = float(1.0 / (128.0 ** 0.5))


def _ln(x, g, b):
    m = jnp.mean(x, axis=-1, keepdims=True)
    xc = x - m
    v = jnp.mean(xc * xc, axis=-1, keepdims=True)
    return xc * jax.lax.rsqrt(v + 1e-5) * g + b


def _softmax_rows(logits):
    m = jnp.max(logits, axis=-1, keepdims=True)
    e = jnp.exp(logits - m)
    return e / jnp.sum(e, axis=-1, keepdims=True)


def _dot(a, b):
    return jnp.dot(a, b, preferred_element_type=jnp.float32)


def _dot_t(a, b):
    return jax.lax.dot_general(a, b, (((1,), (1,)), ((), ())),
                               preferred_element_type=jnp.float32)


def _dot_lt(a, b):
    return jax.lax.dot_general(a, b, (((0,), (0,)), ((), ())),
                               preferred_element_type=jnp.float32)


def _local_kernel(x_ref, wfc_ref, bfc_ref, lwq_ref, lpet_ref, lw_ref,
                  lg_ref, lb_ref, wp_ref, prox_ref, out_ref,
                  h_ref, q_ref, a_ref, m_ref, pls_ref):
    row = jax.lax.broadcasted_iota(jnp.int32, (RL, 1), 0)
    rmask = (row % SP) < S
    h_ref[:] = jnp.zeros((RL, H), jnp.float32)
    wfc = wfc_ref[:]
    bfc = bfc_ref[:]
    for s in range(SAMP):
        xw = x_ref[s]
        for c in range(C):
            o = (s * C + c) * SP
            xc = xw[:, c * IN_DIM:(c + 1) * IN_DIM]
            h_ref[o:o + S, :] = _dot(xc, wfc) + bfc

    col = jax.lax.broadcasted_iota(jnp.int32, (RL, SP), 1)
    cmask = col < S
    for l in range(L_LOC):
        q_ref[:] = _dot(h_ref[:] + lpet_ref[l], lwq_ref[l])

        for g in range(SAMP * C):
            qc = q_ref[g * SP:(g + 1) * SP, :]
            a_ref[g * SP:(g + 1) * SP, :] = _dot_t(qc, qc)

        sim = jnp.where(cmask, a_ref[:] * RSQ_H, jnp.float32(-1e9))
        work = sim
        for _ in range(KNN - 1):
            mx = jnp.max(work, axis=-1, keepdims=True)
            work = jnp.where(work >= mx, jnp.float32(-1e30), work)
        thr = jnp.max(work, axis=-1, keepdims=True)
        a_ref[:] = _softmax_rows(jnp.where(sim >= thr, sim, jnp.float32(-1e9)))

        out = h_ref[:]
        q_ref[:] = out
        dk = 1.0
        for d in range(DEPTH):
            for g in range(SAMP * C):
                sl = slice(g * SP, (g + 1) * SP)
                m_ref[sl, :] = _dot(a_ref[sl, :], q_ref[sl, :])
            cur = jnp.maximum(_dot(m_ref[:], lw_ref[l, d]), 0.0)
            q_ref[:] = cur
            dk *= DECAY
            out = out + dk * cur
        h_ref[:] = jnp.where(rmask, _ln(out, lg_ref[l], lb_ref[l]), 0.0)

    plog = _dot_t(_dot(h_ref[:], wp_ref[:]), prox_ref[:]) * RSQ_H
    pls_ref[:] = jnp.where(rmask, plog, jnp.float32(-1e9))

    for s in range(SAMP):
        for c in range(C):
            g = s * C + c
            sl = slice(g * SP, (g + 1) * SP)
            plc = pls_ref[sl, :]
            pm = jnp.max(plc, axis=0, keepdims=True)
            ex = jnp.exp(plc - pm)
            sc = ex / jnp.sum(ex, axis=0, keepdims=True)
            out_ref[s, c * P:(c + 1) * P, :] = _dot_lt(sc, h_ref[sl, :])


def _global_kernel(hn_ref, gwq_ref, gpet_ref, gw_ref, gg_ref, gb_ref,
                   fw1_ref, fb1_ref, fw2_ref, fb2_ref, fg_ref, fb_ref,
                   out_ref, q_ref, a_ref, m_ref):
    hn = hn_ref[0]
    for l in range(L_GLOB):
        q_ref[:] = _dot(hn + gpet_ref[l], gwq_ref[l])

        for s in range(GB):
            qs = q_ref[s * N:(s + 1) * N, :]
            a_ref[s * N:(s + 1) * N, :] = _dot_t(qs, qs)

        a_ref[:] = _softmax_rows(a_ref[:] * RSQ_H)
        out = hn
        q_ref[:] = hn
        for d in range(DEPTH):
            for s in range(GB):
                sl = slice(s * N, (s + 1) * N)
                m_ref[sl, :] = _dot(a_ref[sl, :], q_ref[sl, :])
            cur = jnp.maximum(_dot(m_ref[:], gw_ref[l, d]), 0.0)
            q_ref[:] = cur
            out = out + cur
        hn = _ln(out, gg_ref[l], gb_ref[l])
    ffn = _dot(jax.nn.gelu(_dot(hn, fw1_ref[:]) + fb1_ref[:]), fw2_ref[:])
    m_ref[:] = _ln(hn + ffn + fb2_ref[:], fg_ref[:], fb_ref[:])

    ii = jax.lax.broadcasted_iota(jnp.int32, (CP, N), 0)
    jj = jax.lax.broadcasted_iota(jnp.int32, (CP, N), 1)
    grp = jnp.where((jj // P == ii) & (ii < C), jnp.float32(1.0 / P),
                    jnp.float32(0.0))

    for s in range(GB):
        zs = m_ref[s * N:(s + 1) * N, :]
        out_ref[0, s * CP:(s + 1) * CP, :] = jnp.maximum(_dot(grp, zs), 0.0)


def _dec_kernel(z_ref, w1_ref, b1_ref, w2_ref, b2_ref, out_ref):
    a = jnp.maximum(_dot(z_ref[:], w1_ref[:]) + b1_ref[:], 0.0)
    out_ref[:, :] = _dot(a, w2_ref[:]) + b2_ref[:]


def _full(shape):
    nd = len(shape)
    return pl.BlockSpec(shape, lambda *_, _nd=nd: (0,) * _nd)


def kernel(x, p, y, params):
    f32 = jnp.float32
    xf = x.reshape(BS, S, C * IN_DIM)
    lpet = jnp.tile(jnp.pad(params['loc_pe'], ((0, 0), (0, SP - S), (0, 0))),
                    (1, SAMP * C, 1))
    ws1 = (params['W_fc'], params['b_fc'].reshape(1, H), params['loc_Wq'],
           lpet, params['loc_W'], params['loc_ln_g'].reshape(L_LOC, 1, H),
           params['loc_ln_b'].reshape(L_LOC, 1, H), params['Wp'],
           params['proxies'])
    in_specs = [pl.BlockSpec((SAMP, S, C * IN_DIM), lambda b: (b, 0, 0))]
    in_specs += [_full(w.shape) for w in ws1]
    pooled = pl.pallas_call(
        _local_kernel,
        grid=(BS // SAMP,),
        in_specs=in_specs,
        out_specs=pl.BlockSpec((SAMP, N, H), lambda b: (b, 0, 0)),
        out_shape=jax.ShapeDtypeStruct((BS, N, H), f32),
        scratch_shapes=[pltpu.VMEM((RL, H), f32), pltpu.VMEM((RL, H), f32),
                        pltpu.VMEM((RL, SP), f32), pltpu.VMEM((RL, H), f32),
                        pltpu.VMEM((RL, P), f32)],
        compiler_params=pltpu.CompilerParams(
            dimension_semantics=("arbitrary",)),
    )(xf, *ws1)

    hn_in = pooled.reshape(BS // GB, GB * N, H)
    gpet = jnp.tile(params['glob_pe'], (1, GB, 1))
    ws2 = (params['glob_Wq'], gpet, params['glob_W'],
           params['glob_ln_g'].reshape(L_GLOB, 1, H),
           params['glob_ln_b'].reshape(L_GLOB, 1, H),
           params['ffn_W1'], params['ffn_b1'].reshape(1, 4 * H),
           params['ffn_W2'], params['ffn_b2'].reshape(1, H),
           params['ffn_ln_g'].reshape(1, H), params['ffn_ln_b'].reshape(1, H))
    in_specs2 = [pl.BlockSpec((1, GB * N, H), lambda b: (b, 0, 0))]
    in_specs2 += [_full(w.shape) for w in ws2]
    z = pl.pallas_call(
        _global_kernel,
        grid=(BS // GB,),
        in_specs=in_specs2,
        out_specs=pl.BlockSpec((1, GB * CP, H), lambda b: (b, 0, 0)),
        out_shape=jax.ShapeDtypeStruct((BS // GB, GB * CP, H), f32),
        scratch_shapes=[pltpu.VMEM((GB * N, H), f32),
                        pltpu.VMEM((GB * N, N), f32),
                        pltpu.VMEM((GB * N, H), f32)],
        compiler_params=pltpu.CompilerParams(
            dimension_semantics=("arbitrary",)),
    )(hn_in, *ws2)

    zf = z.reshape(BS, CP, H)[:, :C, :].reshape(BS, C * H)
    return pl.pallas_call(
        _dec_kernel,
        in_specs=[_full((BS, C * H)), _full((C * H, H)), _full((1, H)),
                  _full((H, NCLS)), _full((1, NCLS))],
        out_specs=_full((BS, NCLS)),
        out_shape=jax.ShapeDtypeStruct((BS, NCLS), f32),
    )(zf, params['dec_W1'], params['dec_b1'].reshape(1, H),
      params['dec_W2'], params['dec_b2'].reshape(1, NCLS))
